# baseline (speedup 1.0000x reference)
import functools
import math

import jax
import jax.numpy as jnp
from jax import lax
from jax.experimental import pallas as pl
from jax.experimental.pallas import tpu as pltpu
from jax.experimental.pallas import tpu_sc as plsc

LP = 40
NW = 32
GB = 4
LANES = 16


def _r_body(subs_ref, wq_ref, wk_ref, sc_ref, out_ref, *, bB):
    subs = subs_ref[...]
    S = subs.shape[1]
    sc_vec = sc_ref[0, 0, :]
    sel = (lax.broadcasted_iota(jnp.int32, (bB, S), 1) ==
           sc_vec[:, None]).astype(subs.dtype)
    q = jnp.einsum('bs,bsd->bd', sel, subs,
                   preferred_element_type=jnp.float32)
    qh = jnp.dot(q, wq_ref[...], preferred_element_type=jnp.float32)
    out_ref[...] = lax.dot_general(qh, wk_ref[...], (((1,), (1,)), ((), ())),
                                   preferred_element_type=jnp.float32)


def _compute_r(substation_embeddings, sc, Wq, Wk):
    B, S, D = substation_embeddings.shape
    bB = 256
    nblk = B // bB
    sc3 = sc.reshape(nblk, 1, bB)
    return pl.pallas_call(
        functools.partial(_r_body, bB=bB),
        grid=(nblk,),
        in_specs=[
            pl.BlockSpec((bB, S, D), lambda i: (i, 0, 0)),
            pl.BlockSpec((D, D), lambda i: (0, 0)),
            pl.BlockSpec((D, D), lambda i: (0, 0)),
            pl.BlockSpec((1, 1, bB), lambda i: (i, 0, 0)),
        ],
        out_specs=pl.BlockSpec((bB, D), lambda i: (i, 0)),
        out_shape=jax.ShapeDtypeStruct((B, D), jnp.float32),
    )(substation_embeddings, Wq, Wk, sc3)


def _sc_body(node_flat, idx_hbm, r_hbm, lens_hbm, out_hbm,
             idx_v, r_v, lens_v, rows_a, rows_b, scores_v, sems,
             *, nb_tile, D, inv_sqrt):
    wid = lax.axis_index("s") * 2 + lax.axis_index("c")
    base = wid * nb_tile
    ngroups = nb_tile // GB
    rows_g = GB * LP

    pltpu.sync_copy(idx_hbm.at[pl.ds(base * LP, nb_tile * LP)], idx_v)
    pltpu.sync_copy(r_hbm.at[pl.ds(base * D, nb_tile * D)], r_v)
    pltpu.sync_copy(lens_hbm.at[pl.ds(base, nb_tile)], lens_v)

    bufs = (rows_a, rows_b)

    def gather(g):
        buf = bufs[g % 2]
        return pltpu.make_async_copy(
            node_flat.at[idx_v.at[pl.ds(g * rows_g, rows_g)]],
            buf, sems.at[g % 2])

    gather(0).start()
    for g in range(ngroups):
        if g + 1 < ngroups:
            gather(g + 1).start()
        gather(g).wait()
        buf = bufs[g % 2]

        @pl.loop(0, GB)
        def _batch(j):
            i_loc = g * GB + j
            roff = i_loc * D
            rc = [r_v[pl.ds(roff + c * LANES, LANES)] for c in range(D // LANES)]
            for cc, clen in ((0, LANES), (1, LANES), (2, LP - 2 * LANES)):
                @pl.loop(0, clen, init_carry=jnp.zeros((LANES,), jnp.float32))
                def _row(ki, acc):
                    row = j * LP + cc * LANES + ki
                    p = buf[row, pl.ds(0, LANES)] * rc[0]
                    for c in range(1, D // LANES):
                        p = p + buf[row, pl.ds(c * LANES, LANES)] * rc[c]
                    s = jnp.sum(p)
                    lane = lax.iota(jnp.int32, LANES)
                    return acc + jnp.where(lane == ki, s, 0.0)

                scores_v[pl.ds(i_loc * LP + cc * LANES, LANES)] = _row

    @pl.loop(0, (nb_tile * LP) // LANES)
    def _sig(c16):
        pos0 = c16 * LANES
        v = scores_v[pl.ds(pos0, LANES)]
        pos = pos0 + lax.iota(jnp.int32, LANES)
        b_loc = lax.div(pos, LP)
        k = pos - b_loc * LP
        lensl = plsc.load_gather(lens_v, [b_loc])
        sig = 1.0 / (1.0 + jnp.exp(v * (-inv_sqrt)))
        scores_v[pl.ds(pos0, LANES)] = jnp.where(k < lensl, sig, 0.0)

    pltpu.sync_copy(scores_v.at[pl.ds(0, nb_tile * LP)],
                    out_hbm.at[pl.ds(base * LP, nb_tile * LP)])


def kernel(node_embeddings, substation_embeddings, sub_choice, Wq, Wk,
           elem_idx_table, elem_lengths):
    B, N, D = node_embeddings.shape
    L = elem_idx_table.shape[1]
    nb_tile = B // NW

    sc = sub_choice[:, 0].astype(jnp.int32)
    starts = elem_idx_table[:, 0][sc].astype(jnp.int32)
    lens = elem_lengths[sc].astype(jnp.int32)
    row_idx = (jnp.arange(B, dtype=jnp.int32)[:, None] * N + starts[:, None] +
               jnp.arange(LP, dtype=jnp.int32)[None, :]).reshape(-1)

    r = _compute_r(substation_embeddings, sc, Wq, Wk).reshape(-1)

    mesh = plsc.VectorSubcoreMesh(core_axis_name="c", subcore_axis_name="s")
    sc_fn = pl.kernel(
        functools.partial(_sc_body, nb_tile=nb_tile, D=D,
                          inv_sqrt=1.0 / math.sqrt(D)),
        out_type=jax.ShapeDtypeStruct((B * LP,), jnp.float32),
        mesh=mesh,
        compiler_params=pltpu.CompilerParams(needs_layout_passes=False),
        scratch_types=[
            pltpu.VMEM((nb_tile * LP,), jnp.int32),
            pltpu.VMEM((nb_tile * D,), jnp.float32),
            pltpu.VMEM((nb_tile,), jnp.int32),
            pltpu.VMEM((GB * LP, D), jnp.float32),
            pltpu.VMEM((GB * LP, D), jnp.float32),
            pltpu.VMEM((nb_tile * LP + LANES,), jnp.float32),
            pltpu.SemaphoreType.DMA((2,)),
        ],
    )
    out_flat = sc_fn(node_embeddings.reshape(B * N, D), row_idx, r, lens)

    busbar_one_logits = out_flat.reshape(B, LP)[:, :L][:, None, :]
    return busbar_one_logits, sub_choice


# speedup vs baseline: 1.0459x; 1.0459x over previous
import functools
import math

import jax
import jax.numpy as jnp
from jax import lax
from jax.experimental import pallas as pl
from jax.experimental.pallas import tpu as pltpu
from jax.experimental.pallas import tpu_sc as plsc

LP = 40
NW = 32
GB = 4
LANES = 16


def _r_body(subs_ref, wq_ref, wk_ref, sc_ref, out_ref, *, bB):
    subs = subs_ref[...]
    S = subs.shape[1]
    sc_vec = sc_ref[0, 0, :]
    sel = (lax.broadcasted_iota(jnp.int32, (bB, S), 1) ==
           sc_vec[:, None]).astype(subs.dtype)
    q = jnp.einsum('bs,bsd->bd', sel, subs,
                   preferred_element_type=jnp.float32)
    qh = jnp.dot(q, wq_ref[...], preferred_element_type=jnp.float32)
    out_ref[...] = lax.dot_general(qh, wk_ref[...], (((1,), (1,)), ((), ())),
                                   preferred_element_type=jnp.float32)


def _compute_r(substation_embeddings, sc, Wq, Wk):
    B, S, D = substation_embeddings.shape
    bB = 256
    nblk = B // bB
    sc3 = sc.reshape(nblk, 1, bB)
    return pl.pallas_call(
        functools.partial(_r_body, bB=bB),
        grid=(nblk,),
        in_specs=[
            pl.BlockSpec((bB, S, D), lambda i: (i, 0, 0)),
            pl.BlockSpec((D, D), lambda i: (0, 0)),
            pl.BlockSpec((D, D), lambda i: (0, 0)),
            pl.BlockSpec((1, 1, bB), lambda i: (i, 0, 0)),
        ],
        out_specs=pl.BlockSpec((bB, D), lambda i: (i, 0)),
        out_shape=jax.ShapeDtypeStruct((B, D), jnp.float32),
    )(substation_embeddings, Wq, Wk, sc3)


def _sc_body(node_flat, idx_hbm, r_hbm, lens_hbm, out_hbm,
             idx_v, r_v, lens_v, rows_a, rows_b, scores_v, sems,
             *, nb_tile, D, inv_sqrt):
    wid = lax.axis_index("s") * 2 + lax.axis_index("c")
    base = wid * nb_tile
    ngroups = nb_tile // GB
    rows_g = GB * LP

    pltpu.sync_copy(idx_hbm.at[pl.ds(base * LP, nb_tile * LP)], idx_v)
    pltpu.sync_copy(r_hbm.at[pl.ds(base * D, nb_tile * D)], r_v)
    pltpu.sync_copy(lens_hbm.at[pl.ds(base, nb_tile)], lens_v)

    bufs = (rows_a, rows_b)
    lane = lax.iota(jnp.int32, LANES)

    def gather(g, buf, sem_slot):
        return pltpu.make_async_copy(
            node_flat.at[idx_v.at[pl.ds(g * rows_g, rows_g)]],
            buf, sems.at[sem_slot])

    gather(0, rows_a, 0).start()

    @pl.loop(0, ngroups, step=2, unroll=1)
    def _grp(g0):
        for half in (0, 1):
            g = g0 + half
            buf = bufs[half]

            @pl.when(g + 1 < ngroups)
            def _():
                gather(g + 1, bufs[1 - half], 1 - half).start()

            gather(g, buf, half).wait()

            @pl.loop(0, GB, unroll=1)
            def _batch(j):
                i_loc = g * GB + j
                roff = i_loc * D
                rc = [r_v[pl.ds(roff + c * LANES, LANES)]
                      for c in range(D // LANES)]
                lens16 = lens_v[pl.ds(lax.div(i_loc, LANES) * LANES, LANES)]
                blen = jnp.sum(jnp.where(lane == lax.rem(i_loc, LANES),
                                         lens16, 0))
                nchunk = lax.div(blen + (LANES - 1), LANES)

                @pl.loop(0, nchunk, unroll=1)
                def _chunk(cc):
                    clen = jnp.minimum(blen - cc * LANES, LANES)

                    @pl.loop(0, clen, unroll=1,
                             init_carry=jnp.zeros((LANES,), jnp.float32))
                    def _row(ki, acc):
                        row = j * LP + cc * LANES + ki
                        p = buf[row, pl.ds(0, LANES)] * rc[0]
                        for c in range(1, D // LANES):
                            p = p + buf[row, pl.ds(c * LANES, LANES)] * rc[c]
                        s = jnp.sum(p)
                        return acc + jnp.where(lane == ki, s, 0.0)

                    scores_v[pl.ds(i_loc * LP + cc * LANES, LANES)] = _row

    @pl.loop(0, (nb_tile * LP) // LANES)
    def _sig(c16):
        pos0 = c16 * LANES
        v = scores_v[pl.ds(pos0, LANES)]
        pos = pos0 + lax.iota(jnp.int32, LANES)
        b_loc = lax.div(pos, LP)
        k = pos - b_loc * LP
        lensl = plsc.load_gather(lens_v, [b_loc])
        sig = 1.0 / (1.0 + jnp.exp(v * (-inv_sqrt)))
        scores_v[pl.ds(pos0, LANES)] = jnp.where(k < lensl, sig, 0.0)

    pltpu.sync_copy(scores_v.at[pl.ds(0, nb_tile * LP)],
                    out_hbm.at[pl.ds(base * LP, nb_tile * LP)])


def kernel(node_embeddings, substation_embeddings, sub_choice, Wq, Wk,
           elem_idx_table, elem_lengths):
    B, N, D = node_embeddings.shape
    L = elem_idx_table.shape[1]
    nb_tile = B // NW

    sc = sub_choice[:, 0].astype(jnp.int32)
    starts = elem_idx_table[:, 0][sc].astype(jnp.int32)
    lens = elem_lengths[sc].astype(jnp.int32)
    row_idx = (jnp.arange(B, dtype=jnp.int32)[:, None] * N + starts[:, None] +
               jnp.arange(LP, dtype=jnp.int32)[None, :]).reshape(-1)

    r = _compute_r(substation_embeddings, sc, Wq, Wk).reshape(-1)

    mesh = plsc.VectorSubcoreMesh(core_axis_name="c", subcore_axis_name="s")
    sc_fn = pl.kernel(
        functools.partial(_sc_body, nb_tile=nb_tile, D=D,
                          inv_sqrt=1.0 / math.sqrt(D)),
        out_type=jax.ShapeDtypeStruct((B * LP,), jnp.float32),
        mesh=mesh,
        compiler_params=pltpu.CompilerParams(needs_layout_passes=False),
        scratch_types=[
            pltpu.VMEM((nb_tile * LP,), jnp.int32),
            pltpu.VMEM((nb_tile * D,), jnp.float32),
            pltpu.VMEM((nb_tile,), jnp.int32),
            pltpu.VMEM((GB * LP, D), jnp.float32),
            pltpu.VMEM((GB * LP, D), jnp.float32),
            pltpu.VMEM((nb_tile * LP + LANES,), jnp.float32),
            pltpu.SemaphoreType.DMA((2,)),
        ],
    )
    out_flat = sc_fn(node_embeddings.reshape(B * N, D), row_idx, r, lens)

    busbar_one_logits = out_flat.reshape(B, LP)[:, :L][:, None, :]
    return busbar_one_logits, sub_choice


# speedup vs baseline: 1.0653x; 1.0186x over previous
import functools
import math

import jax
import jax.numpy as jnp
from jax import lax
from jax.experimental import pallas as pl
from jax.experimental.pallas import tpu as pltpu
from jax.experimental.pallas import tpu_sc as plsc

LP = 40
NW = 32
GB = 8
LANES = 16


def _r_body(subs_ref, wq_ref, wk_ref, sc_ref, out_ref, *, bB):
    subs = subs_ref[...]
    S = subs.shape[1]
    sc_vec = sc_ref[0, 0, :]
    sel = (lax.broadcasted_iota(jnp.int32, (bB, S), 1) ==
           sc_vec[:, None]).astype(subs.dtype)
    q = jnp.einsum('bs,bsd->bd', sel, subs,
                   preferred_element_type=jnp.float32)
    qh = jnp.dot(q, wq_ref[...], preferred_element_type=jnp.float32)
    out_ref[...] = lax.dot_general(qh, wk_ref[...], (((1,), (1,)), ((), ())),
                                   preferred_element_type=jnp.float32)


def _compute_r(substation_embeddings, sc, Wq, Wk):
    B, S, D = substation_embeddings.shape
    bB = 256
    nblk = B // bB
    sc3 = sc.reshape(nblk, 1, bB)
    return pl.pallas_call(
        functools.partial(_r_body, bB=bB),
        grid=(nblk,),
        in_specs=[
            pl.BlockSpec((bB, S, D), lambda i: (i, 0, 0)),
            pl.BlockSpec((D, D), lambda i: (0, 0)),
            pl.BlockSpec((D, D), lambda i: (0, 0)),
            pl.BlockSpec((1, 1, bB), lambda i: (i, 0, 0)),
        ],
        out_specs=pl.BlockSpec((bB, D), lambda i: (i, 0)),
        out_shape=jax.ShapeDtypeStruct((B, D), jnp.float32),
    )(substation_embeddings, Wq, Wk, sc3)


def _sc_body(node_flat, idx_hbm, r_hbm, lens_hbm, out_hbm,
             idx_v, r_v, lens_v, rows_a, rows_b, scores_v, sems,
             *, nb_tile, D, inv_sqrt):
    wid = lax.axis_index("s") * 2 + lax.axis_index("c")
    base = wid * nb_tile
    ngroups = nb_tile // GB
    rows_g = GB * LP

    pltpu.sync_copy(idx_hbm.at[pl.ds(base * LP, nb_tile * LP)], idx_v)
    pltpu.sync_copy(r_hbm.at[pl.ds(base * D, nb_tile * D)], r_v)
    pltpu.sync_copy(lens_hbm.at[pl.ds(base, nb_tile)], lens_v)

    bufs = (rows_a, rows_b)
    lane = lax.iota(jnp.int32, LANES)

    def gather(g, buf, sem_slot):
        return pltpu.make_async_copy(
            node_flat.at[idx_v.at[pl.ds(g * rows_g, rows_g)]],
            buf.at[pl.ds(0, rows_g), :], sems.at[sem_slot])

    gather(0, rows_a, 0).start()

    @pl.loop(0, ngroups, step=2, unroll=1)
    def _grp(g0):
        for half in (0, 1):
            g = g0 + half
            buf = bufs[half]

            @pl.when(g + 1 < ngroups)
            def _():
                gather(g + 1, bufs[1 - half], 1 - half).start()

            gather(g, buf, half).wait()

            @pl.loop(0, GB, unroll=1)
            def _batch(j):
                i_loc = g * GB + j
                roff = i_loc * D
                rc = [r_v[pl.ds(roff + c * LANES, LANES)]
                      for c in range(D // LANES)]
                lens16 = lens_v[pl.ds(lax.div(i_loc, LANES) * LANES, LANES)]
                blen = jnp.sum(jnp.where(lane == lax.rem(i_loc, LANES),
                                         lens16, 0))
                nchunk = lax.div(blen + (LANES - 1), LANES)

                @pl.loop(0, nchunk, unroll=1)
                def _chunk(cc):
                    @pl.loop(0, LANES, unroll=4,
                             init_carry=jnp.zeros((LANES,), jnp.float32))
                    def _row(ki, acc):
                        row = j * LP + cc * LANES + ki
                        terms = [buf[row, pl.ds(c * LANES, LANES)] * rc[c]
                                 for c in range(D // LANES)]
                        while len(terms) > 1:
                            terms = [a + b for a, b in
                                     zip(terms[::2], terms[1::2])]
                        s = jnp.sum(terms[0])
                        return acc + jnp.where(lane == ki, s, 0.0)

                    scores_v[pl.ds(i_loc * LP + cc * LANES, LANES)] = _row

    @pl.loop(0, (nb_tile * LP) // LANES)
    def _sig(c16):
        pos0 = c16 * LANES
        v = scores_v[pl.ds(pos0, LANES)]
        pos = pos0 + lax.iota(jnp.int32, LANES)
        b_loc = lax.div(pos, LP)
        k = pos - b_loc * LP
        lensl = plsc.load_gather(lens_v, [b_loc])
        sig = 1.0 / (1.0 + jnp.exp(v * (-inv_sqrt)))
        scores_v[pl.ds(pos0, LANES)] = jnp.where(k < lensl, sig, 0.0)

    pltpu.sync_copy(scores_v.at[pl.ds(0, nb_tile * LP)],
                    out_hbm.at[pl.ds(base * LP, nb_tile * LP)])


def kernel(node_embeddings, substation_embeddings, sub_choice, Wq, Wk,
           elem_idx_table, elem_lengths):
    B, N, D = node_embeddings.shape
    L = elem_idx_table.shape[1]
    nb_tile = B // NW

    sc = sub_choice[:, 0].astype(jnp.int32)
    starts = elem_idx_table[:, 0][sc].astype(jnp.int32)
    lens = elem_lengths[sc].astype(jnp.int32)
    row_idx = (jnp.arange(B, dtype=jnp.int32)[:, None] * N + starts[:, None] +
               jnp.arange(LP, dtype=jnp.int32)[None, :]).reshape(-1)

    r = _compute_r(substation_embeddings, sc, Wq, Wk).reshape(-1)

    mesh = plsc.VectorSubcoreMesh(core_axis_name="c", subcore_axis_name="s")
    sc_fn = pl.kernel(
        functools.partial(_sc_body, nb_tile=nb_tile, D=D,
                          inv_sqrt=1.0 / math.sqrt(D)),
        out_type=jax.ShapeDtypeStruct((B * LP,), jnp.float32),
        mesh=mesh,
        compiler_params=pltpu.CompilerParams(needs_layout_passes=False),
        scratch_types=[
            pltpu.VMEM((nb_tile * LP,), jnp.int32),
            pltpu.VMEM((nb_tile * D,), jnp.float32),
            pltpu.VMEM((nb_tile,), jnp.int32),
            pltpu.VMEM((GB * LP + 8, D), jnp.float32),
            pltpu.VMEM((GB * LP + 8, D), jnp.float32),
            pltpu.VMEM((nb_tile * LP + LANES,), jnp.float32),
            pltpu.SemaphoreType.DMA((2,)),
        ],
    )
    out_flat = sc_fn(node_embeddings.reshape(B * N, D), row_idx, r, lens)

    busbar_one_logits = out_flat.reshape(B, LP)[:, :L][:, None, :]
    return busbar_one_logits, sub_choice


# speedup vs baseline: 1.0991x; 1.0317x over previous
import functools
import math

import jax
import jax.numpy as jnp
from jax import lax
from jax.experimental import pallas as pl
from jax.experimental.pallas import tpu as pltpu
from jax.experimental.pallas import tpu_sc as plsc

LP = 40
NW = 32
GB = 4
NBUF = 4
LANES = 16


def _r_body(subs_ref, wq_ref, wk_ref, sc_ref, out_ref, *, bB):
    subs = subs_ref[...]
    S = subs.shape[1]
    sc_vec = sc_ref[0, 0, :]
    sel = (lax.broadcasted_iota(jnp.int32, (bB, S), 1) ==
           sc_vec[:, None]).astype(subs.dtype)
    q = jnp.einsum('bs,bsd->bd', sel, subs,
                   preferred_element_type=jnp.float32)
    qh = jnp.dot(q, wq_ref[...], preferred_element_type=jnp.float32)
    out_ref[...] = lax.dot_general(qh, wk_ref[...], (((1,), (1,)), ((), ())),
                                   preferred_element_type=jnp.float32)


def _compute_r(substation_embeddings, sc, Wq, Wk):
    B, S, D = substation_embeddings.shape
    bB = 256
    nblk = B // bB
    sc3 = sc.reshape(nblk, 1, bB)
    return pl.pallas_call(
        functools.partial(_r_body, bB=bB),
        grid=(nblk,),
        in_specs=[
            pl.BlockSpec((bB, S, D), lambda i: (i, 0, 0)),
            pl.BlockSpec((D, D), lambda i: (0, 0)),
            pl.BlockSpec((D, D), lambda i: (0, 0)),
            pl.BlockSpec((1, 1, bB), lambda i: (i, 0, 0)),
        ],
        out_specs=pl.BlockSpec((bB, D), lambda i: (i, 0)),
        out_shape=jax.ShapeDtypeStruct((B, D), jnp.float32),
    )(substation_embeddings, Wq, Wk, sc3)


def _sc_body(node_flat, idx_hbm, r_hbm, lens_hbm, out_hbm,
             idx_v, r_v, lens_v, bufs, scores_v, sems,
             *, nb_tile, D, inv_sqrt):
    wid = lax.axis_index("s") * 2 + lax.axis_index("c")
    base = wid * nb_tile
    ngroups = nb_tile // GB
    nbuf = len(bufs)
    rows_g = GB * LP

    pltpu.sync_copy(idx_hbm.at[pl.ds(base * LP, nb_tile * LP)], idx_v)
    pltpu.sync_copy(r_hbm.at[pl.ds(base * D, nb_tile * D)], r_v)
    pltpu.sync_copy(lens_hbm.at[pl.ds(base, nb_tile)], lens_v)

    lane = lax.iota(jnp.int32, LANES)

    def gather(g, slot):
        return pltpu.make_async_copy(
            node_flat.at[idx_v.at[pl.ds(g * rows_g, rows_g)]],
            bufs[slot].at[pl.ds(0, rows_g), :], sems.at[slot])

    for p in range(nbuf - 1):
        gather(p, p).start()

    @pl.loop(0, ngroups, step=nbuf, unroll=1)
    def _grp(g0):
        for half in range(nbuf):
            g = g0 + half
            buf = bufs[half]

            @pl.when(g + nbuf - 1 < ngroups)
            def _():
                gather(g + nbuf - 1, (half + nbuf - 1) % nbuf).start()

            gather(g, half).wait()

            @pl.loop(0, GB, unroll=1)
            def _batch(j):
                i_loc = g * GB + j
                roff = i_loc * D
                rc = [r_v[pl.ds(roff + c * LANES, LANES)]
                      for c in range(D // LANES)]
                lens16 = lens_v[pl.ds(lax.div(i_loc, LANES) * LANES, LANES)]
                blen = jnp.sum(jnp.where(lane == lax.rem(i_loc, LANES),
                                         lens16, 0))
                nchunk = lax.div(blen + (LANES - 1), LANES)

                @pl.loop(0, nchunk, unroll=1)
                def _chunk(cc):
                    @pl.loop(0, LANES, unroll=4,
                             init_carry=jnp.zeros((LANES,), jnp.float32))
                    def _row(ki, acc):
                        row = j * LP + cc * LANES + ki
                        terms = [buf[row, pl.ds(c * LANES, LANES)] * rc[c]
                                 for c in range(D // LANES)]
                        while len(terms) > 1:
                            terms = [a + b for a, b in
                                     zip(terms[::2], terms[1::2])]
                        s = jnp.sum(terms[0])
                        return acc + jnp.where(lane == ki, s, 0.0)

                    scores_v[pl.ds(i_loc * LP + cc * LANES, LANES)] = _row

    @pl.loop(0, (nb_tile * LP) // LANES)
    def _sig(c16):
        pos0 = c16 * LANES
        v = scores_v[pl.ds(pos0, LANES)]
        pos = pos0 + lax.iota(jnp.int32, LANES)
        b_loc = lax.div(pos, LP)
        k = pos - b_loc * LP
        lensl = plsc.load_gather(lens_v, [b_loc])
        sig = 1.0 / (1.0 + jnp.exp(v * (-inv_sqrt)))
        scores_v[pl.ds(pos0, LANES)] = jnp.where(k < lensl, sig, 0.0)

    pltpu.sync_copy(scores_v.at[pl.ds(0, nb_tile * LP)],
                    out_hbm.at[pl.ds(base * LP, nb_tile * LP)])


def kernel(node_embeddings, substation_embeddings, sub_choice, Wq, Wk,
           elem_idx_table, elem_lengths):
    B, N, D = node_embeddings.shape
    L = elem_idx_table.shape[1]
    nb_tile = B // NW

    sc = sub_choice[:, 0].astype(jnp.int32)
    starts = elem_idx_table[:, 0][sc].astype(jnp.int32)
    lens = elem_lengths[sc].astype(jnp.int32)
    row_idx = (jnp.arange(B, dtype=jnp.int32)[:, None] * N + starts[:, None] +
               jnp.arange(LP, dtype=jnp.int32)[None, :]).reshape(-1)

    r = _compute_r(substation_embeddings, sc, Wq, Wk).reshape(-1)

    mesh = plsc.VectorSubcoreMesh(core_axis_name="c", subcore_axis_name="s")
    sc_fn = pl.kernel(
        functools.partial(_sc_body, nb_tile=nb_tile, D=D,
                          inv_sqrt=1.0 / math.sqrt(D)),
        out_type=jax.ShapeDtypeStruct((B * LP,), jnp.float32),
        mesh=mesh,
        compiler_params=pltpu.CompilerParams(needs_layout_passes=False),
        scratch_types=[
            pltpu.VMEM((nb_tile * LP,), jnp.int32),
            pltpu.VMEM((nb_tile * D,), jnp.float32),
            pltpu.VMEM((nb_tile,), jnp.int32),
            tuple(pltpu.VMEM((GB * LP + 8, D), jnp.float32)
                  for _ in range(NBUF)),
            pltpu.VMEM((nb_tile * LP + LANES,), jnp.float32),
            pltpu.SemaphoreType.DMA((NBUF,)),
        ],
    )
    out_flat = sc_fn(node_embeddings.reshape(B * N, D), row_idx, r, lens)

    busbar_one_logits = out_flat.reshape(B, LP)[:, :L][:, None, :]
    return busbar_one_logits, sub_choice


# speedup vs baseline: 1.1227x; 1.0215x over previous
import functools
import math

import jax
import jax.numpy as jnp
from jax import lax
from jax.experimental import pallas as pl
from jax.experimental.pallas import tpu as pltpu
from jax.experimental.pallas import tpu_sc as plsc

LP = 40
NW = 32
GB = 4
NBUF = 4
LANES = 16


def _r_body(subs_ref, wq_ref, wk_ref, sc_ref, out_ref, *, bB):
    subs = subs_ref[...]
    S = subs.shape[1]
    sc_vec = sc_ref[0, 0, :]
    sel = (lax.broadcasted_iota(jnp.int32, (bB, S), 1) ==
           sc_vec[:, None]).astype(subs.dtype)
    q = jnp.einsum('bs,bsd->bd', sel, subs,
                   preferred_element_type=jnp.float32)
    w = lax.dot_general(wq_ref[...], wk_ref[...], (((1,), (1,)), ((), ())),
                        preferred_element_type=jnp.float32)
    out_ref[...] = jnp.dot(q, w, preferred_element_type=jnp.float32)


def _compute_r(substation_embeddings, sc, Wq, Wk):
    B, S, D = substation_embeddings.shape
    bB = B
    sc3 = sc.reshape(1, 1, bB)
    return pl.pallas_call(
        functools.partial(_r_body, bB=bB),
        grid=(1,),
        in_specs=[
            pl.BlockSpec((bB, S, D), lambda i: (0, 0, 0)),
            pl.BlockSpec((D, D), lambda i: (0, 0)),
            pl.BlockSpec((D, D), lambda i: (0, 0)),
            pl.BlockSpec((1, 1, bB), lambda i: (0, 0, 0)),
        ],
        out_specs=pl.BlockSpec((bB, D), lambda i: (0, 0)),
        out_shape=jax.ShapeDtypeStruct((B, D), jnp.float32),
    )(substation_embeddings, Wq, Wk, sc3)


def _sc_body(node_flat, idx_hbm, r_hbm, lens_hbm, out_hbm,
             idx_v, r_v, lens_v, bufs, scores_v, sems,
             *, nb_tile, D, inv_sqrt):
    wid = lax.axis_index("s") * 2 + lax.axis_index("c")
    base = wid * nb_tile
    ngroups = nb_tile // GB
    nbuf = len(bufs)
    rows_g = GB * LP

    pltpu.sync_copy(idx_hbm.at[pl.ds(base * LP, nb_tile * LP)], idx_v)
    pltpu.sync_copy(r_hbm.at[pl.ds(base * D, nb_tile * D)], r_v)
    pltpu.sync_copy(lens_hbm.at[pl.ds(base, nb_tile)], lens_v)

    lane = lax.iota(jnp.int32, LANES)

    def gather(g, slot):
        return pltpu.make_async_copy(
            node_flat.at[idx_v.at[pl.ds(g * rows_g, rows_g)]],
            bufs[slot].at[pl.ds(0, rows_g), :], sems.at[slot])

    for p in range(nbuf - 1):
        gather(p, p).start()

    @pl.loop(0, ngroups, step=nbuf, unroll=1)
    def _grp(g0):
        for half in range(nbuf):
            g = g0 + half
            buf = bufs[half]

            @pl.when(g + nbuf - 1 < ngroups)
            def _():
                gather(g + nbuf - 1, (half + nbuf - 1) % nbuf).start()

            gather(g, half).wait()

            @pl.loop(0, GB, unroll=1)
            def _batch(j):
                i_loc = g * GB + j
                roff = i_loc * D
                rc = [r_v[pl.ds(roff + c * LANES, LANES)]
                      for c in range(D // LANES)]
                lens16 = lens_v[pl.ds(lax.div(i_loc, LANES) * LANES, LANES)]
                blen = jnp.sum(jnp.where(lane == lax.rem(i_loc, LANES),
                                         lens16, 0))
                nchunk = lax.div(blen + (LANES - 1), LANES)

                @pl.loop(0, nchunk, unroll=1)
                def _chunk(cc):
                    @pl.loop(0, LANES, unroll=4,
                             init_carry=jnp.zeros((LANES,), jnp.float32))
                    def _row(ki, acc):
                        row = j * LP + cc * LANES + ki
                        terms = [buf[row, pl.ds(c * LANES, LANES)] * rc[c]
                                 for c in range(D // LANES)]
                        while len(terms) > 1:
                            terms = [a + b for a, b in
                                     zip(terms[::2], terms[1::2])]
                        s = jnp.sum(terms[0])
                        return acc + jnp.where(lane == ki, s, 0.0)

                    scores_v[pl.ds(i_loc * LP + cc * LANES, LANES)] = _row

    @pl.loop(0, (nb_tile * LP) // LANES)
    def _sig(c16):
        pos0 = c16 * LANES
        v = scores_v[pl.ds(pos0, LANES)]
        pos = pos0 + lax.iota(jnp.int32, LANES)
        b_loc = lax.div(pos, LP)
        k = pos - b_loc * LP
        lensl = plsc.load_gather(lens_v, [b_loc])
        sig = 1.0 / (1.0 + jnp.exp(v * (-inv_sqrt)))
        scores_v[pl.ds(pos0, LANES)] = jnp.where(k < lensl, sig, 0.0)

    pltpu.sync_copy(scores_v.at[pl.ds(0, nb_tile * LP)],
                    out_hbm.at[pl.ds(base * LP, nb_tile * LP)])


def kernel(node_embeddings, substation_embeddings, sub_choice, Wq, Wk,
           elem_idx_table, elem_lengths):
    B, N, D = node_embeddings.shape
    L = elem_idx_table.shape[1]
    nb_tile = B // NW

    sc = sub_choice[:, 0].astype(jnp.int32)
    starts = elem_idx_table[:, 0][sc].astype(jnp.int32)
    lens = elem_lengths[sc].astype(jnp.int32)
    row_idx = (jnp.arange(B, dtype=jnp.int32)[:, None] * N + starts[:, None] +
               jnp.arange(LP, dtype=jnp.int32)[None, :]).reshape(-1)

    r = _compute_r(substation_embeddings, sc, Wq, Wk).reshape(-1)

    mesh = plsc.VectorSubcoreMesh(core_axis_name="c", subcore_axis_name="s")
    sc_fn = pl.kernel(
        functools.partial(_sc_body, nb_tile=nb_tile, D=D,
                          inv_sqrt=1.0 / math.sqrt(D)),
        out_type=jax.ShapeDtypeStruct((B * LP,), jnp.float32),
        mesh=mesh,
        compiler_params=pltpu.CompilerParams(needs_layout_passes=False),
        scratch_types=[
            pltpu.VMEM((nb_tile * LP,), jnp.int32),
            pltpu.VMEM((nb_tile * D,), jnp.float32),
            pltpu.VMEM((nb_tile,), jnp.int32),
            tuple(pltpu.VMEM((GB * LP + 8, D), jnp.float32)
                  for _ in range(NBUF)),
            pltpu.VMEM((nb_tile * LP + LANES,), jnp.float32),
            pltpu.SemaphoreType.DMA((NBUF,)),
        ],
    )
    out_flat = sc_fn(node_embeddings.reshape(B * N, D), row_idx, r, lens)

    busbar_one_logits = out_flat.reshape(B, LP)[:, :L][:, None, :]
    return busbar_one_logits, sub_choice


# speedup vs baseline: 1.1340x; 1.0101x over previous
import functools
import math

import jax
import jax.numpy as jnp
from jax import lax
from jax.experimental import pallas as pl
from jax.experimental.pallas import tpu as pltpu
from jax.experimental.pallas import tpu_sc as plsc

LP = 40
NW = 32
GB = 4
NBUF = 4
LANES = 16


def _r_body(subs_ref, wq_ref, wk_ref, sc_ref, out_ref, *, bB):
    subs = subs_ref[...]
    S = subs.shape[1]
    sc_vec = sc_ref[0, 0, :]
    sel = (lax.broadcasted_iota(jnp.int32, (bB, S), 1) ==
           sc_vec[:, None]).astype(subs.dtype)
    q = jnp.einsum('bs,bsd->bd', sel, subs,
                   preferred_element_type=jnp.float32)
    w = lax.dot_general(wq_ref[...], wk_ref[...], (((1,), (1,)), ((), ())),
                        preferred_element_type=jnp.float32)
    out_ref[...] = jnp.dot(q, w, preferred_element_type=jnp.float32)


def _compute_r(substation_embeddings, sc, Wq, Wk):
    B, S, D = substation_embeddings.shape
    bB = B
    sc3 = sc.reshape(1, 1, bB)
    return pl.pallas_call(
        functools.partial(_r_body, bB=bB),
        grid=(1,),
        in_specs=[
            pl.BlockSpec((bB, S, D), lambda i: (0, 0, 0)),
            pl.BlockSpec((D, D), lambda i: (0, 0)),
            pl.BlockSpec((D, D), lambda i: (0, 0)),
            pl.BlockSpec((1, 1, bB), lambda i: (0, 0, 0)),
        ],
        out_specs=pl.BlockSpec((bB, D), lambda i: (0, 0)),
        out_shape=jax.ShapeDtypeStruct((B, D), jnp.float32),
    )(substation_embeddings, Wq, Wk, sc3)


def _sc_body(node_flat, idx_hbm, r_hbm, lens_hbm, out_hbm,
             idx_v, r_v, lens_v, bufs, scores_v, sems,
             *, nb_tile, D, inv_sqrt):
    wid = lax.axis_index("s") * 2 + lax.axis_index("c")
    base = wid * nb_tile
    ngroups = nb_tile // GB
    nbuf = len(bufs)
    rows_g = GB * LP

    pltpu.sync_copy(idx_hbm.at[pl.ds(base * LP, nb_tile * LP)], idx_v)
    pltpu.sync_copy(r_hbm.at[pl.ds(base * D, nb_tile * D)], r_v)
    pltpu.sync_copy(lens_hbm.at[pl.ds(base, nb_tile)], lens_v)

    lane = lax.iota(jnp.int32, LANES)

    half_g = rows_g // 2

    def gather_parts(g, slot):
        return [pltpu.make_async_copy(
            node_flat.at[idx_v.at[pl.ds(g * rows_g + h * half_g, half_g)]],
            bufs[slot].at[pl.ds(h * half_g, half_g), :], sems.at[slot, h])
            for h in (0, 1)]

    def start(g, slot):
        for c in gather_parts(g, slot):
            c.start()

    def wait(g, slot):
        for c in gather_parts(g, slot):
            c.wait()

    for p in range(nbuf - 1):
        start(p, p)

    @pl.loop(0, ngroups, step=nbuf, unroll=1)
    def _grp(g0):
        for half in range(nbuf):
            g = g0 + half
            buf = bufs[half]

            @pl.when(g + nbuf - 1 < ngroups)
            def _():
                start(g + nbuf - 1, (half + nbuf - 1) % nbuf)

            wait(g, half)

            @pl.loop(0, GB, unroll=1)
            def _batch(j):
                i_loc = g * GB + j
                roff = i_loc * D
                rc = [r_v[pl.ds(roff + c * LANES, LANES)]
                      for c in range(D // LANES)]
                lens16 = lens_v[pl.ds(lax.div(i_loc, LANES) * LANES, LANES)]
                blen = jnp.sum(jnp.where(lane == lax.rem(i_loc, LANES),
                                         lens16, 0))
                nchunk = lax.div(blen + (LANES - 1), LANES)

                @pl.loop(0, nchunk, unroll=1)
                def _chunk(cc):
                    @pl.loop(0, LANES, unroll=4,
                             init_carry=jnp.zeros((LANES,), jnp.float32))
                    def _row(ki, acc):
                        row = j * LP + cc * LANES + ki
                        terms = [buf[row, pl.ds(c * LANES, LANES)] * rc[c]
                                 for c in range(D // LANES)]
                        while len(terms) > 1:
                            terms = [a + b for a, b in
                                     zip(terms[::2], terms[1::2])]
                        s = jnp.sum(terms[0])
                        return acc + jnp.where(lane == ki, s, 0.0)

                    scores_v[pl.ds(i_loc * LP + cc * LANES, LANES)] = _row

    @pl.loop(0, (nb_tile * LP) // LANES)
    def _sig(c16):
        pos0 = c16 * LANES
        v = scores_v[pl.ds(pos0, LANES)]
        pos = pos0 + lax.iota(jnp.int32, LANES)
        b_loc = lax.div(pos, LP)
        k = pos - b_loc * LP
        lensl = plsc.load_gather(lens_v, [b_loc])
        sig = 1.0 / (1.0 + jnp.exp(v * (-inv_sqrt)))
        scores_v[pl.ds(pos0, LANES)] = jnp.where(k < lensl, sig, 0.0)

    pltpu.sync_copy(scores_v.at[pl.ds(0, nb_tile * LP)],
                    out_hbm.at[pl.ds(base * LP, nb_tile * LP)])


def kernel(node_embeddings, substation_embeddings, sub_choice, Wq, Wk,
           elem_idx_table, elem_lengths):
    B, N, D = node_embeddings.shape
    L = elem_idx_table.shape[1]
    nb_tile = B // NW

    sc = sub_choice[:, 0].astype(jnp.int32)
    starts = elem_idx_table[:, 0][sc].astype(jnp.int32)
    lens = elem_lengths[sc].astype(jnp.int32)
    row_idx = (jnp.arange(B, dtype=jnp.int32)[:, None] * N + starts[:, None] +
               jnp.arange(LP, dtype=jnp.int32)[None, :]).reshape(-1)

    r = _compute_r(substation_embeddings, sc, Wq, Wk).reshape(-1)

    mesh = plsc.VectorSubcoreMesh(core_axis_name="c", subcore_axis_name="s")
    sc_fn = pl.kernel(
        functools.partial(_sc_body, nb_tile=nb_tile, D=D,
                          inv_sqrt=1.0 / math.sqrt(D)),
        out_type=jax.ShapeDtypeStruct((B * LP,), jnp.float32),
        mesh=mesh,
        compiler_params=pltpu.CompilerParams(needs_layout_passes=False),
        scratch_types=[
            pltpu.VMEM((nb_tile * LP,), jnp.int32),
            pltpu.VMEM((nb_tile * D,), jnp.float32),
            pltpu.VMEM((nb_tile,), jnp.int32),
            tuple(pltpu.VMEM((GB * LP + 8, D), jnp.float32)
                  for _ in range(NBUF)),
            pltpu.VMEM((nb_tile * LP + LANES,), jnp.float32),
            pltpu.SemaphoreType.DMA((NBUF, 2)),
        ],
    )
    out_flat = sc_fn(node_embeddings.reshape(B * N, D), row_idx, r, lens)

    busbar_one_logits = out_flat.reshape(B, LP)[:, :L][:, None, :]
    return busbar_one_logits, sub_choice
